# baseline (speedup 1.0000x reference)
import functools

import jax
import jax.numpy as jnp
from jax import lax
from jax.experimental import pallas as pl
from jax.experimental.pallas import tpu as pltpu
from jax.experimental.pallas import tpu_sc as plsc

NC = 2
NS = 16
NW = NC * NS
LANES = 16
CHUNK = 32


def _gather_body(tok_w, nch, hid, ids_hbm, tok_hbm, gath_hbm, idx_v,
                 r0, r1, b0, b1, gsem, osem):
    rows = [r0, r1]
    bfs = [b0, b1]
    ngr = hid // (2 * LANES)
    wid = lax.axis_index("s") * NC + lax.axis_index("c")
    base = wid * tok_w

    pltpu.sync_copy(ids_hbm.at[wid], idx_v)
    i2 = jnp.arange(0, 2 * LANES, 2, dtype=jnp.int32)

    def start_gather(k, s):
        pltpu.async_copy(tok_hbm.at[idx_v.at[k]], rows[s], gsem[s])

    def out_slice(k):
        return gath_hbm.at[pl.ds(base + k * CHUNK, CHUNK)]

    start_gather(0, 0)
    start_gather(1, 1)
    for k in range(nch):
        s = k % 2
        pltpu.make_async_copy(tok_hbm.at[idx_v.at[k]], rows[s], gsem[s]).wait()
        if k >= 2:
            pltpu.make_async_copy(bfs[s], out_slice(k - 2), osem[s]).wait()

        def token_body(t, _):
            tv = jnp.full((LANES,), t, jnp.int32)
            for g in range(ngr):
                ev = plsc.load_gather(rows[s], [tv, i2 + (2 * LANES * g)])
                od = plsc.load_gather(rows[s], [tv, i2 + (2 * LANES * g + 1)])
                pk = plsc.pack(ev, od, format=plsc.PackFormat.INTERLEAVED)
                w = plsc.bitcast(pk, jnp.int32)
                bfs[s][t, pl.ds(g * LANES, LANES)] = w
            return ()

        lax.fori_loop(0, CHUNK, token_body, (), unroll=False)

        pltpu.async_copy(bfs[s], out_slice(k), osem[s])
        if k + 2 < nch:
            start_gather(k + 2, s)
    for k in range(max(0, nch - 2), nch):
        pltpu.make_async_copy(bfs[k % 2], out_slice(k), osem[k % 2]).wait()


def _sc_gather(ids, tok_table):
    nw_tok = ids.shape[0] * ids.shape[1] * ids.shape[2] // NW
    nch = nw_tok // CHUNK
    hid = tok_table.shape[1]
    mesh = plsc.VectorSubcoreMesh(core_axis_name="c", subcore_axis_name="s",
                                  num_cores=NC, num_subcores=NS)
    run = pl.kernel(
        functools.partial(_gather_body, nw_tok, nch, hid),
        out_type=jax.ShapeDtypeStruct((NW * nw_tok, hid // 2), jnp.int32),
        mesh=mesh,
        scratch_types=[
            pltpu.VMEM((nch, CHUNK), jnp.int32),
            pltpu.VMEM((CHUNK, hid), jnp.float32),
            pltpu.VMEM((CHUNK, hid), jnp.float32),
            pltpu.VMEM((CHUNK, hid // 2), jnp.int32),
            pltpu.VMEM((CHUNK, hid // 2), jnp.int32),
            [pltpu.SemaphoreType.DMA] * 2,
            [pltpu.SemaphoreType.DMA] * 2,
        ],
        compiler_params=pltpu.CompilerParams(needs_layout_passes=False),
    )
    return run(ids, tok_table)


def _ln_block(emb_ref, pos_ref, g_ref, b_ref, out_ref):
    x = emb_ref[...].astype(jnp.float32) + pos_ref[...]
    mean = jnp.mean(x, axis=1, keepdims=True)
    xc = x - mean
    var = jnp.mean(xc * xc, axis=1, keepdims=True)
    y = xc * lax.rsqrt(var + 1e-12)
    out_ref[...] = y * g_ref[...] + b_ref[...]


def _tc_layernorm(emb, pos_table, gamma, beta, blk):
    tot, hid = emb.shape
    s = pos_table.shape[0]
    bps = s // blk
    grid = (tot // blk,)
    return pl.pallas_call(
        _ln_block,
        grid=grid,
        in_specs=[
            pl.BlockSpec((blk, hid), lambda i: (i, 0)),
            pl.BlockSpec((blk, hid), lambda i: (lax.rem(i, bps), 0)),
            pl.BlockSpec((1, hid), lambda i: (0, 0)),
            pl.BlockSpec((1, hid), lambda i: (0, 0)),
        ],
        out_specs=pl.BlockSpec((blk, hid), lambda i: (i, 0)),
        out_shape=jax.ShapeDtypeStruct((tot, hid), jnp.float32),
    )(emb, pos_table, gamma.reshape(1, hid), beta.reshape(1, hid))


def kernel(input_ids, tok_table, pos_table, gamma, beta):
    b, s = input_ids.shape
    hid = tok_table.shape[1]
    tot = b * s
    tok_w = tot // NW
    nch = tok_w // CHUNK

    ids = input_ids.astype(jnp.int32).reshape(NW, nch, CHUNK)
    emb_words = _sc_gather(ids, tok_table)
    emb = lax.bitcast_convert_type(emb_words, jnp.bfloat16).reshape(tot, hid)
    out = _tc_layernorm(emb, pos_table, gamma, beta, 1024)
    return out.reshape(b, s, hid)


# speedup vs baseline: 3.1320x; 3.1320x over previous
import functools

import jax
import jax.numpy as jnp
from jax import lax
from jax.experimental import pallas as pl
from jax.experimental.pallas import tpu as pltpu
from jax.experimental.pallas import tpu_sc as plsc

NC = 2
NS = 16
NW = NC * NS
CHUNK = 32
NBUF = 4


def _gather_body(tok_w, nch, ids_hbm, tok_hbm, gath_hbm, idx_v,
                 r0, r1, r2, r3, gsem, osem):
    rows = [r0, r1, r2, r3]
    wid = lax.axis_index("s") * NC + lax.axis_index("c")
    base = wid * tok_w

    pltpu.sync_copy(ids_hbm.at[wid], idx_v)

    def start_gather(k, s):
        pltpu.async_copy(tok_hbm.at[idx_v.at[k]], rows[s], gsem[s])

    def out_slice(k):
        return gath_hbm.at[pl.ds(base + k * CHUNK, CHUNK)]

    start_gather(0, 0)
    start_gather(1, 1)
    for k in range(nch):
        s = k % NBUF
        pltpu.make_async_copy(tok_hbm.at[idx_v.at[k]], rows[s], gsem[s]).wait()
        pltpu.async_copy(rows[s], out_slice(k), osem[s])
        if k + 2 < nch:
            s2 = (k + 2) % NBUF
            if k >= 2:
                pltpu.make_async_copy(rows[s2], out_slice(k - 2), osem[s2]).wait()
            start_gather(k + 2, s2)
    for k in range(max(0, nch - NBUF), nch):
        s = k % NBUF
        pltpu.make_async_copy(rows[s], out_slice(k), osem[s]).wait()


def _sc_gather(ids, tok_table):
    nw_tok = ids.shape[0] * ids.shape[1] * ids.shape[2] // NW
    nch = nw_tok // CHUNK
    hid = tok_table.shape[1]
    mesh = plsc.VectorSubcoreMesh(core_axis_name="c", subcore_axis_name="s",
                                  num_cores=NC, num_subcores=NS)
    run = pl.kernel(
        functools.partial(_gather_body, nw_tok, nch),
        out_type=jax.ShapeDtypeStruct((NW * nw_tok, hid), jnp.float32),
        mesh=mesh,
        scratch_types=[
            pltpu.VMEM((nch, CHUNK), jnp.int32),
            pltpu.VMEM((CHUNK, hid), jnp.float32),
            pltpu.VMEM((CHUNK, hid), jnp.float32),
            pltpu.VMEM((CHUNK, hid), jnp.float32),
            pltpu.VMEM((CHUNK, hid), jnp.float32),
            [pltpu.SemaphoreType.DMA] * NBUF,
            [pltpu.SemaphoreType.DMA] * NBUF,
        ],
        compiler_params=pltpu.CompilerParams(needs_layout_passes=False),
    )
    return run(ids, tok_table)


def _ln_block(emb_ref, pos_ref, g_ref, b_ref, out_ref):
    x = emb_ref[...] + pos_ref[...]
    mean = jnp.mean(x, axis=1, keepdims=True)
    xc = x - mean
    var = jnp.mean(xc * xc, axis=1, keepdims=True)
    y = xc * lax.rsqrt(var + 1e-12)
    out_ref[...] = y * g_ref[...] + b_ref[...]


def _tc_layernorm(emb, pos_table, gamma, beta, blk):
    tot, hid = emb.shape
    s = pos_table.shape[0]
    bps = s // blk
    grid = (tot // blk,)
    return pl.pallas_call(
        _ln_block,
        grid=grid,
        in_specs=[
            pl.BlockSpec((blk, hid), lambda i: (i, 0)),
            pl.BlockSpec((blk, hid), lambda i: (lax.rem(i, bps), 0)),
            pl.BlockSpec((1, hid), lambda i: (0, 0)),
            pl.BlockSpec((1, hid), lambda i: (0, 0)),
        ],
        out_specs=pl.BlockSpec((blk, hid), lambda i: (i, 0)),
        out_shape=jax.ShapeDtypeStruct((tot, hid), jnp.float32),
    )(emb, pos_table, gamma.reshape(1, hid), beta.reshape(1, hid))


def kernel(input_ids, tok_table, pos_table, gamma, beta):
    b, s = input_ids.shape
    hid = tok_table.shape[1]
    tot = b * s
    tok_w = tot // NW
    nch = tok_w // CHUNK

    ids = input_ids.astype(jnp.int32).reshape(NW, nch, CHUNK)
    emb = _sc_gather(ids, tok_table)
    out = _tc_layernorm(emb, pos_table, gamma, beta, 2048)
    return out.reshape(b, s, hid)


# speedup vs baseline: 3.1597x; 1.0088x over previous
import functools

import jax
import jax.numpy as jnp
from jax import lax
from jax.experimental import pallas as pl
from jax.experimental.pallas import tpu as pltpu
from jax.experimental.pallas import tpu_sc as plsc

NC = 2
NS = 16
NW = NC * NS
CHUNK = 64
NBUF = 2


def _gather_body(tok_w, nch, ids_hbm, tok_hbm, gath_hbm, idx_v,
                 r0, r1, gsem, osem):
    rows = [r0, r1]
    wid = lax.axis_index("s") * NC + lax.axis_index("c")
    base = wid * tok_w

    pltpu.sync_copy(ids_hbm.at[wid], idx_v)

    def start_gather(k, s):
        pltpu.async_copy(tok_hbm.at[idx_v.at[k]], rows[s], gsem[s])

    def out_slice(k):
        return gath_hbm.at[pl.ds(base + k * CHUNK, CHUNK)]

    start_gather(0, 0)
    start_gather(1, 1)
    for k in range(nch):
        s = k % NBUF
        pltpu.make_async_copy(tok_hbm.at[idx_v.at[k]], rows[s], gsem[s]).wait()
        pltpu.async_copy(rows[s], out_slice(k), osem[s])
        if k + 2 < nch:
            pltpu.make_async_copy(rows[s], out_slice(k), osem[s]).wait()
            start_gather(k + 2, s)
    for k in range(max(0, nch - NBUF), nch):
        s = k % NBUF
        pltpu.make_async_copy(rows[s], out_slice(k), osem[s]).wait()


def _sc_gather(ids, tok_table):
    nw_tok = ids.shape[0] * ids.shape[1] * ids.shape[2] // NW
    nch = nw_tok // CHUNK
    hid = tok_table.shape[1]
    mesh = plsc.VectorSubcoreMesh(core_axis_name="c", subcore_axis_name="s",
                                  num_cores=NC, num_subcores=NS)
    run = pl.kernel(
        functools.partial(_gather_body, nw_tok, nch),
        out_type=jax.ShapeDtypeStruct((NW * nw_tok, hid), jnp.float32),
        mesh=mesh,
        scratch_types=[
            pltpu.VMEM((nch, CHUNK), jnp.int32),
            pltpu.VMEM((CHUNK, hid), jnp.float32),
            pltpu.VMEM((CHUNK, hid), jnp.float32),
            [pltpu.SemaphoreType.DMA] * NBUF,
            [pltpu.SemaphoreType.DMA] * NBUF,
        ],
        compiler_params=pltpu.CompilerParams(needs_layout_passes=False),
    )
    return run(ids, tok_table)


def _ln_block(emb_ref, pos_ref, g_ref, b_ref, out_ref):
    x = emb_ref[...] + pos_ref[...]
    mean = jnp.mean(x, axis=1, keepdims=True)
    xc = x - mean
    var = jnp.mean(xc * xc, axis=1, keepdims=True)
    y = xc * lax.rsqrt(var + 1e-12)
    out_ref[...] = y * g_ref[...] + b_ref[...]


def _tc_layernorm(emb, pos_table, gamma, beta, blk):
    tot, hid = emb.shape
    s = pos_table.shape[0]
    bps = s // blk
    grid = (tot // blk,)
    return pl.pallas_call(
        _ln_block,
        grid=grid,
        in_specs=[
            pl.BlockSpec((blk, hid), lambda i: (i, 0)),
            pl.BlockSpec((blk, hid), lambda i: (lax.rem(i, bps), 0)),
            pl.BlockSpec((1, hid), lambda i: (0, 0)),
            pl.BlockSpec((1, hid), lambda i: (0, 0)),
        ],
        out_specs=pl.BlockSpec((blk, hid), lambda i: (i, 0)),
        out_shape=jax.ShapeDtypeStruct((tot, hid), jnp.float32),
    )(emb, pos_table, gamma.reshape(1, hid), beta.reshape(1, hid))


def kernel(input_ids, tok_table, pos_table, gamma, beta):
    b, s = input_ids.shape
    hid = tok_table.shape[1]
    tot = b * s
    tok_w = tot // NW
    nch = tok_w // CHUNK

    ids = input_ids.astype(jnp.int32).reshape(NW, nch, CHUNK)
    emb = _sc_gather(ids, tok_table)
    out = _tc_layernorm(emb, pos_table, gamma, beta, 2048)
    return out.reshape(b, s, hid)
